# baseline (speedup 1.0000x reference)
import functools

import jax
import jax.numpy as jnp
from jax import lax
from jax.experimental import pallas as pl
from jax.experimental.pallas import tpu as pltpu
from jax.experimental.pallas import tpu_sc as plsc

K = 8
G = 1024
NEG = float(-3.402823e38)


def _seg_ids_kernel(in_ref, ids_ref, m_ref, *, nc):
    i = pl.program_id(1)
    x2 = in_ref[0]
    g2 = G // 2
    while x2.shape[0] > g2:
        x2 = jnp.maximum(x2[: x2.shape[0] // 2], x2[x2.shape[0] // 2:])

    @pl.when(i == 0)
    def _init():
        m_ref[...] = x2

    @pl.when(i > 0)
    def _fold():
        m_ref[...] = jnp.maximum(m_ref[...], x2)

    @pl.when(i == nc - 1)
    def _extract():
        m2 = m_ref[...]
        c = m2.shape[1] // 2
        m = jnp.concatenate([m2[:, :c], m2[:, c:]], axis=0)
        iota = lax.broadcasted_iota(jnp.int32, m.shape, 0)
        ids = []
        for _ in range(K):
            mx = jnp.max(m, axis=0, keepdims=True)
            idx = jnp.where(m == mx, iota, G)
            fi = jnp.min(idx, axis=0, keepdims=True)
            m = jnp.where(iota == fi, NEG, m)
            sigma = jnp.where(fi < g2, 2 * fi, 2 * (fi - g2) + 1)
            ids.append(sigma)
        ids_ref[0] = jnp.concatenate(ids, axis=0)


def _phase1(packed, b, s, c):
    r2 = 2048
    nc = (s // 2) // r2
    return pl.pallas_call(
        functools.partial(_seg_ids_kernel, nc=nc),
        grid=(b, nc),
        in_specs=[pl.BlockSpec((1, r2, 2 * c), lambda bi, si: (bi, si, 0))],
        out_specs=pl.BlockSpec((1, K, c), lambda bi, si: (bi, 0, 0)),
        out_shape=jax.ShapeDtypeStruct((b, K, c), jnp.int32),
        scratch_shapes=[pltpu.VMEM((G // 2, 2 * c), jnp.float32)],
        compiler_params=pltpu.CompilerParams(
            dimension_semantics=("arbitrary", "arbitrary"),
        ),
    )(packed)


def _make_sc_gather(b, s, c):
    info = plsc.get_sparse_core_info()
    nw = info.num_cores * info.num_subcores
    el = s // G
    pairs = b * c
    ppw = pairs // nw
    cand = K * el
    epw = ppw * cand
    rows = epw // 128
    bsz = s * c
    mesh = plsc.VectorSubcoreMesh(core_axis_name="c", subcore_axis_name="s")

    @functools.partial(
        pl.kernel,
        mesh=mesh,
        out_type=jax.ShapeDtypeStruct((nw, rows, 128), jnp.float32),
        scratch_types=[
            pltpu.VMEM((ppw * K,), jnp.int32),
            pltpu.VMEM((rows, 128), jnp.int32),
            pltpu.VMEM((rows, 128), jnp.float32),
            pltpu.SemaphoreType.DMA,
        ],
    )
    def sc_gather(flat_hbm, ids_hbm, out_hbm, ids_v, idx_v, dst_v, sem):
        w = lax.axis_index("s") * info.num_cores + lax.axis_index("c")
        pltpu.sync_copy(ids_hbm.at[pl.ds(w * ppw * K, ppw * K)], ids_v)
        lane = lax.iota(jnp.int32, 16)
        vlane = lane * (G * c)
        for pp in range(ppw // 2):
            v = ids_v[pl.ds(pp * 16, 16)]
            for half in range(2):
                p = pp * 2 + half
                pair = w * ppw + p
                pb = pair // c
                pc = pair % c
                base = pb * bsz + pc
                for j in range(K):
                    sg = v[half * K + j]
                    sj = base + (sg >> 1) * (2 * c) + (sg & 1) * c
                    for tg in range(el // 16):
                        e = p * cand + j * el + tg * 16
                        idx_v[e // 128, pl.ds(e % 128, 16)] = (
                            vlane + (sj + tg * (16 * G * c)))
        for r0 in range(0, rows, 32):
            copies = [
                pltpu.async_copy(flat_hbm.at[idx_v.at[i]], dst_v.at[i], sem)
                for i in range(r0, min(r0 + 32, rows))
            ]
            for cp in copies:
                cp.wait()
        pltpu.sync_copy(dst_v, out_hbm.at[w])

    return sc_gather


def _final_kernel(in_ref, out_ref):
    x = in_ref[0]
    rows = x.shape[0]
    iota = lax.broadcasted_iota(jnp.int32, x.shape, 0)
    outs = []
    for _ in range(K):
        m = jnp.max(x, axis=0, keepdims=True)
        idx = jnp.where(x == m, iota, rows)
        fi = jnp.min(idx, axis=0, keepdims=True)
        x = jnp.where(iota == fi, NEG, x)
        outs.append(m)
    out_ref[0] = jnp.concatenate(outs, axis=0)


def _phase3(cands):
    b, cand, c = cands.shape
    return pl.pallas_call(
        _final_kernel,
        grid=(b,),
        in_specs=[pl.BlockSpec((1, cand, c), lambda bi: (bi, 0, 0))],
        out_specs=pl.BlockSpec((1, K, c), lambda bi: (bi, 0, 0)),
        out_shape=jax.ShapeDtypeStruct((b, K, c), jnp.float32),
    )(cands)


@jax.jit
def kernel(inputs):
    b, s, c = inputs.shape
    el = s // G
    packed = jnp.reshape(inputs, (b, s // 2, 2 * c))
    ids = _phase1(packed, b, s, c)
    ids_t = jnp.transpose(ids, (0, 2, 1))
    flat_in = jnp.reshape(packed, (-1,))
    cands = _make_sc_gather(b, s, c)(flat_in, jnp.reshape(ids_t, (-1,)))
    cands = jnp.reshape(cands, (b, c, K * el))
    cands = jnp.transpose(cands, (0, 2, 1))
    return _phase3(cands)


# speedup vs baseline: 1.7190x; 1.7190x over previous
import functools

import jax
import jax.numpy as jnp
from jax import lax
from jax.experimental import pallas as pl
from jax.experimental.pallas import tpu as pltpu
from jax.experimental.pallas import tpu_sc as plsc

K = 8
G = 512
NEG = float(-3.402823e38)


def _seg_ids_kernel(in_ref, ids_ref, packed_ref, m_ref, *, nc):
    i = pl.program_id(1)
    x = in_ref[0]
    h = x.shape[0] // 2
    x2 = jnp.concatenate([x[:h], x[h:]], axis=1)
    packed_ref[...] = x2.reshape(-1)
    while x2.shape[0] > G:
        x2 = jnp.maximum(x2[: x2.shape[0] // 2], x2[x2.shape[0] // 2:])

    @pl.when(i == 0)
    def _init():
        m_ref[...] = x2

    @pl.when(i > 0)
    def _fold():
        m_ref[...] = jnp.maximum(m_ref[...], x2)

    @pl.when(i == nc - 1)
    def _extract():
        m2 = m_ref[...]
        c = m2.shape[1] // 2
        m = jnp.maximum(m2[:, :c], m2[:, c:])
        iota = lax.broadcasted_iota(jnp.int32, m.shape, 0)
        ids = []
        for _ in range(K):
            mx = jnp.max(m, axis=0, keepdims=True)
            idx = jnp.where(m == mx, iota, G)
            fi = jnp.min(idx, axis=0, keepdims=True)
            m = jnp.where(iota == fi, NEG, m)
            ids.append(fi)
        ids_ref[0] = jnp.concatenate(ids, axis=0)


def _phase1(inputs):
    b, s, c = inputs.shape
    r = 8192
    nc = s // r
    ids, packed = pl.pallas_call(
        functools.partial(_seg_ids_kernel, nc=nc),
        grid=(b, nc),
        in_specs=[pl.BlockSpec((1, r, c), lambda bi, si: (bi, si, 0))],
        out_specs=[
            pl.BlockSpec((1, K, c), lambda bi, si: (bi, 0, 0)),
            pl.BlockSpec((r * c,), lambda bi, si: (bi * nc + si,)),
        ],
        out_shape=[
            jax.ShapeDtypeStruct((b, K, c), jnp.int32),
            jax.ShapeDtypeStruct((b * s * c,), jnp.float32),
        ],
        scratch_shapes=[pltpu.VMEM((G, 2 * c), jnp.float32)],
        compiler_params=pltpu.CompilerParams(
            dimension_semantics=("arbitrary", "arbitrary"),
        ),
    )(inputs)
    return ids, packed


def _make_sc_gather(b, s, c):
    info = plsc.get_sparse_core_info()
    nw = info.num_cores * info.num_subcores
    el = s // G
    pairs = b * c
    ppw = pairs // nw
    cand = K * el
    epw = ppw * cand
    rows = epw // 128
    bsz = s * c
    mesh = plsc.VectorSubcoreMesh(core_axis_name="c", subcore_axis_name="s")

    @functools.partial(
        pl.kernel,
        mesh=mesh,
        out_type=jax.ShapeDtypeStruct((nw, rows, 128), jnp.float32),
        scratch_types=[
            pltpu.VMEM((ppw * K,), jnp.int32),
            pltpu.VMEM((rows, 128), jnp.int32),
            pltpu.VMEM((rows, 128), jnp.float32),
            pltpu.SemaphoreType.DMA,
        ],
    )
    def sc_gather(flat_hbm, ids_hbm, out_hbm, ids_v, idx_v, dst_v, sem):
        w = lax.axis_index("s") * info.num_cores + lax.axis_index("c")
        pltpu.sync_copy(ids_hbm.at[pl.ds(w * ppw * K, ppw * K)], ids_v)
        lane = lax.iota(jnp.int32, 16)
        vlane = (((lane >> 3) & 1) * c
                 + (lane & 7) * (G * 2 * c))
        for pp in range(ppw // 2):
            v = ids_v[pl.ds(pp * 16, 16)]
            for half in range(2):
                p = pp * 2 + half
                pair = w * ppw + p
                pb = pair // c
                pc = pair % c
                base = pb * bsz + pc
                for j in range(K):
                    sj = base + v[half * K + j] * (2 * c)
                    for tg in range(el // 16):
                        e = p * cand + j * el + tg * 16
                        idx_v[e // 128, pl.ds(e % 128, 16)] = (
                            vlane + (sj + tg * (4096 * 128)))
        for r0 in range(0, rows, 32):
            copies = [
                pltpu.async_copy(flat_hbm.at[idx_v.at[i]], dst_v.at[i], sem)
                for i in range(r0, r0 + 32)
            ]
            for cp in copies:
                cp.wait()
        pltpu.sync_copy(dst_v, out_hbm.at[w])

    return sc_gather


def _final_kernel(in_ref, out_ref):
    x = in_ref[0]
    rows = x.shape[0]
    iota = lax.broadcasted_iota(jnp.int32, x.shape, 0)
    outs = []
    for _ in range(K):
        m = jnp.max(x, axis=0, keepdims=True)
        idx = jnp.where(x == m, iota, rows)
        fi = jnp.min(idx, axis=0, keepdims=True)
        x = jnp.where(iota == fi, NEG, x)
        outs.append(m)
    out_ref[0] = jnp.concatenate(outs, axis=0)


def _phase3(cands):
    b, cand, c = cands.shape
    return pl.pallas_call(
        _final_kernel,
        grid=(b,),
        in_specs=[pl.BlockSpec((1, cand, c), lambda bi: (bi, 0, 0))],
        out_specs=pl.BlockSpec((1, K, c), lambda bi: (bi, 0, 0)),
        out_shape=jax.ShapeDtypeStruct((b, K, c), jnp.float32),
    )(cands)


@jax.jit
def kernel(inputs):
    b, s, c = inputs.shape
    el = s // G
    ids, packed = _phase1(inputs)
    ids_t = jnp.transpose(ids, (0, 2, 1))
    cands = _make_sc_gather(b, s, c)(packed, jnp.reshape(ids_t, (-1,)))
    cands = jnp.reshape(cands, (b, c, K * el))
    cands = jnp.transpose(cands, (0, 2, 1))
    return _phase3(cands)


# speedup vs baseline: 1.7663x; 1.0275x over previous
import functools

import jax
import jax.numpy as jnp
from jax import lax
from jax.experimental import pallas as pl
from jax.experimental.pallas import tpu as pltpu
from jax.experimental.pallas import tpu_sc as plsc

K = 8
G = 512
NEG = float(-3.402823e38)


def _seg_ids_kernel(in_ref, ids_ref, packed_ref, m_ref, *, nc):
    i = pl.program_id(1)
    x = in_ref[0]
    h = x.shape[0] // 2
    x2 = jnp.concatenate([x[:h], x[h:]], axis=1)
    packed_ref[...] = x2.reshape(-1)
    while x2.shape[0] > G:
        x2 = jnp.maximum(x2[: x2.shape[0] // 2], x2[x2.shape[0] // 2:])

    @pl.when(i == 0)
    def _init():
        m_ref[...] = x2

    @pl.when(i > 0)
    def _fold():
        m_ref[...] = jnp.maximum(m_ref[...], x2)

    @pl.when(i == nc - 1)
    def _extract():
        m2 = m_ref[...]
        c = m2.shape[1] // 2
        m = jnp.maximum(m2[:, :c], m2[:, c:])
        iota = lax.broadcasted_iota(jnp.int32, m.shape, 0)
        ids = []
        for _ in range(K):
            mx = jnp.max(m, axis=0, keepdims=True)
            idx = jnp.where(m == mx, iota, G)
            fi = jnp.min(idx, axis=0, keepdims=True)
            m = jnp.where(iota == fi, NEG, m)
            ids.append(fi)
        ids_ref[0] = jnp.concatenate(ids, axis=0)


def _phase1(inputs):
    b, s, c = inputs.shape
    r = 16384
    nc = s // r
    ids, packed = pl.pallas_call(
        functools.partial(_seg_ids_kernel, nc=nc),
        grid=(b, nc),
        in_specs=[pl.BlockSpec((1, r, c), lambda bi, si: (bi, si, 0))],
        out_specs=[
            pl.BlockSpec((1, K, c), lambda bi, si: (bi, 0, 0)),
            pl.BlockSpec((r * c,), lambda bi, si: (bi * nc + si,)),
        ],
        out_shape=[
            jax.ShapeDtypeStruct((b, K, c), jnp.int32),
            jax.ShapeDtypeStruct((b * s * c,), jnp.float32),
        ],
        scratch_shapes=[pltpu.VMEM((G, 2 * c), jnp.float32)],
        compiler_params=pltpu.CompilerParams(
            dimension_semantics=("arbitrary", "arbitrary"),
        ),
    )(inputs)
    return ids, packed


def _make_sc_gather(b, s, c):
    info = plsc.get_sparse_core_info()
    nw = info.num_cores * info.num_subcores
    el = s // G
    pairs = b * c
    ppw = pairs // nw
    cand = K * el
    epw = ppw * cand
    rows = epw // 128
    bsz = s * c
    mesh = plsc.VectorSubcoreMesh(core_axis_name="c", subcore_axis_name="s")

    @functools.partial(
        pl.kernel,
        mesh=mesh,
        out_type=jax.ShapeDtypeStruct((nw, rows, 128), jnp.float32),
        scratch_types=[
            pltpu.VMEM((ppw * K,), jnp.int32),
            pltpu.VMEM((rows, 128), jnp.int32),
            pltpu.VMEM((rows, 128), jnp.float32),
            pltpu.SemaphoreType.DMA,
        ],
    )
    def sc_gather(flat_hbm, ids_hbm, out_hbm, ids_v, idx_v, dst_v, sem):
        w = lax.axis_index("s") * info.num_cores + lax.axis_index("c")
        pltpu.sync_copy(ids_hbm.at[pl.ds(w * ppw * K, ppw * K)], ids_v)
        lane = lax.iota(jnp.int32, 16)
        vlane = lane * (G * 2 * c)
        for pp in range(ppw // 2):
            v = ids_v[pl.ds(pp * 16, 16)]
            for half in range(2):
                p = pp * 2 + half
                pair = w * ppw + p
                pb = pair // c
                pc = pair % c
                base = pb * bsz + pc
                for j in range(K):
                    sj = base + v[half * K + j] * (2 * c)
                    for tg in range(el // 16):
                        e = p * cand + j * el + tg * 16
                        idx_v[e // 128, pl.ds(e % 128, 16)] = (
                            vlane
                            + (sj + (tg // 2) * (8192 * 128) + (tg % 2) * c))
        for r0 in range(0, rows, 32):
            copies = [
                pltpu.async_copy(flat_hbm.at[idx_v.at[i]], dst_v.at[i], sem)
                for i in range(r0, r0 + 32)
            ]
            for cp in copies:
                cp.wait()
        pltpu.sync_copy(dst_v, out_hbm.at[w])

    return sc_gather


def _final_kernel(in_ref, out_ref):
    x = in_ref[0]
    rows = x.shape[0]
    iota = lax.broadcasted_iota(jnp.int32, x.shape, 0)
    outs = []
    for _ in range(K):
        m = jnp.max(x, axis=0, keepdims=True)
        idx = jnp.where(x == m, iota, rows)
        fi = jnp.min(idx, axis=0, keepdims=True)
        x = jnp.where(iota == fi, NEG, x)
        outs.append(m)
    out_ref[0] = jnp.concatenate(outs, axis=0)


def _phase3(cands):
    b, cand, c = cands.shape
    return pl.pallas_call(
        _final_kernel,
        grid=(b,),
        in_specs=[pl.BlockSpec((1, cand, c), lambda bi: (bi, 0, 0))],
        out_specs=pl.BlockSpec((1, K, c), lambda bi: (bi, 0, 0)),
        out_shape=jax.ShapeDtypeStruct((b, K, c), jnp.float32),
    )(cands)


@jax.jit
def kernel(inputs):
    b, s, c = inputs.shape
    el = s // G
    ids, packed = _phase1(inputs)
    ids_t = jnp.transpose(ids, (0, 2, 1))
    cands = _make_sc_gather(b, s, c)(packed, jnp.reshape(ids_t, (-1,)))
    cands = jnp.reshape(cands, (b, c, K * el))
    cands = jnp.transpose(cands, (0, 2, 1))
    return _phase3(cands)
